# baseline (speedup 1.0000x reference)
import functools

import jax
import jax.numpy as jnp
import numpy as np
from jax import lax
from jax.experimental import pallas as pl
from jax.experimental.pallas import tpu as pltpu
from jax.experimental.pallas import tpu_sc as plsc

N = 16384
D = 256
BS = 4096
SB = 64
NB = N // BS
B_WIN = 20
L_SCALE = np.float32(0.5 * (0.95 + 1.05))


def _batcher_pairs(n):
    m = 1
    while m < n:
        m *= 2
    pairs = []

    def oddeven_merge(lo, hi, r):
        step = r * 2
        if step < hi - lo:
            oddeven_merge(lo, hi, step)
            oddeven_merge(lo + r, hi, step)
            for i in range(lo + r, hi - r, step):
                pairs.append((i, i + r))
        else:
            pairs.append((lo, lo + r))

    def oddeven_sort(lo, hi):
        if hi - lo >= 1:
            mid = lo + (hi - lo) // 2
            oddeven_sort(lo, mid)
            oddeven_sort(mid + 1, hi)
            oddeven_merge(lo, hi, 1)

    oddeven_sort(0, m - 1)
    return [(i, j) for (i, j) in pairs if j < n]


def _median_network(n):
    pairs = _batcher_pairs(n)
    needed = {n // 2 - 1, n // 2}
    keep = []
    for (i, j) in reversed(pairs):
        if i in needed or j in needed:
            keep.append((i, j))
            needed.add(i)
            needed.add(j)
    keep.reverse()
    return keep


_MED_NET = _median_network(2 * B_WIN)

SC_CORES = 2
SC_SUBCORES = 16
SC_WORKERS = SC_CORES * SC_SUBCORES
ROWS_PER_WORKER = N // SC_WORKERS
CH = 64
NCH = ROWS_PER_WORKER // CH


def _rows_down(z, q, fill):
    if q == 0:
        return z
    return jnp.concatenate(
        [jnp.full((q, 128), fill, z.dtype), z[:128 - q, :]], axis=0)


def _rows_up(z, q, fill):
    if q == 0:
        return z
    return jnp.concatenate(
        [z[q:, :], jnp.full((q, 128), fill, z.dtype)], axis=0)


def _shift_down(x, s, fill):
    q, b = s // 128, s % 128
    if b == 0:
        return _rows_down(x, q, fill)
    xq = _rows_down(x, q, fill)
    xq1 = _rows_down(x, q + 1, fill)
    lane = lax.broadcasted_iota(jnp.int32, (128, 128), 1)
    return jnp.where(lane >= b,
                     jnp.roll(xq, b, axis=1),
                     jnp.roll(xq1, b, axis=1))


def _shift_up(x, s, fill):
    q, b = s // 128, s % 128
    if b == 0:
        return _rows_up(x, q, fill)
    xq = _rows_up(x, q, fill)
    xq1 = _rows_up(x, q + 1, fill)
    lane = lax.broadcasted_iota(jnp.int32, (128, 128), 1)
    return jnp.where(lane < 128 - b,
                     jnp.roll(xq, -b, axis=1),
                     jnp.roll(xq1, -b, axis=1))


def _boundary_logic(sim_raw, cu_ref):
    row = lax.broadcasted_iota(jnp.int32, (128, 128), 0)
    lane = lax.broadcasted_iota(jnp.int32, (128, 128), 1)
    pos = row * 128 + lane

    cnt = jnp.zeros((128, 128), jnp.int32)
    for s in range(9):
        cnt = cnt + (cu_ref[s] <= pos).astype(jnp.int32)
    seq = jnp.clip(cnt - 1, 0, 7)
    starts = jnp.zeros((128, 128), jnp.int32)
    ends = jnp.zeros((128, 128), jnp.int32)
    for s in range(8):
        m = seq == s
        starts = jnp.where(m, cu_ref[s], starts)
        ends = jnp.where(m, cu_ref[s + 1] - 1, ends)
    ends = jnp.maximum(ends, starts)
    is_start = pos == starts
    sim = jnp.where(is_start, jnp.float32(1.0), sim_raw)

    sim_end = jnp.zeros((128, 128), jnp.float32)
    for s in range(8):
        e_s = cu_ref[s + 1] - 1
        val = jnp.sum(jnp.where(pos == e_s, sim, 0.0))
        sim_end = jnp.where(seq == s, val, sim_end)

    wins = []
    for off in range(-B_WIN, B_WIN):
        if off == 0:
            sh = sim
        elif off < 0:
            sh = _shift_down(sim, -off, 0.0)
        else:
            sh = _shift_up(sim, off, 0.0)
        ip = pos + off
        w = jnp.where(ip < starts, jnp.float32(1.0),
                      jnp.where(ip > ends, sim_end, sh))
        wins.append(w)

    W = 2 * B_WIN
    arr = list(wins)
    for (i, j) in _MED_NET:
        lo = jnp.minimum(arr[i], arr[j])
        hi = jnp.maximum(arr[i], arr[j])
        arr[i] = lo
        arr[j] = hi
    med = 0.5 * (arr[W // 2 - 1] + arr[W // 2])

    new_start = is_start | (sim < L_SCALE * med)

    x = jnp.where(new_start, pos, -1)
    s = 1
    while s < N:
        x = jnp.maximum(x, _shift_down(x, s, -1))
        s *= 2
    b = x
    y = jnp.where(new_start, pos, N)
    s = 1
    while s < N:
        y = jnp.minimum(y, _shift_up(y, s, N))
        s *= 2
    g = _shift_up(y, 1, N)
    inv = 1.0 / (g - b).astype(jnp.float32)
    return b, g, inv


def _k12_body(flat_ref, tri_ref, cu_ref, e_ref, b_ref, g_ref, inv_ref,
              carry_ref, prev_ref, sim_scr):
    k = pl.program_id(0)

    @pl.when(k == 0)
    def _():
        carry_ref[...] = jnp.zeros_like(carry_ref)

    @pl.when(k < NB)
    def _():
        blk = flat_ref[...]
        n = jnp.sqrt(jnp.sum(blk * blk, axis=1, keepdims=True))
        normed = blk / (n + 1e-8)
        prev = jnp.concatenate([prev_ref[...], normed[:-1, :]], axis=0)
        s = jnp.sum(normed * prev, axis=1)
        sim_scr[pl.ds(k * (BS // 128), BS // 128), :] = s.reshape(BS // 128, 128)
        prev_ref[...] = normed[BS - 1:, :]
        tri = tri_ref[...]
        locs = []
        subsums = []
        for j in range(BS // SB):
            sub = blk[j * SB:(j + 1) * SB, :]
            loc = lax.dot_general(
                tri, sub, (((1,), (0,)), ((), ())),
                preferred_element_type=jnp.float32,
                precision=lax.Precision.HIGHEST)
            locs.append(loc)
            subsums.append(loc[SB - 1:SB, :] + sub[SB - 1:SB, :])
        nsub = BS // SB
        incl = jnp.concatenate(subsums, axis=0)
        s2 = 1
        while s2 < nsub:
            incl = incl + jnp.concatenate(
                [jnp.zeros((s2, D), jnp.float32), incl[:nsub - s2, :]],
                axis=0)
            s2 *= 2
        excl = incl - jnp.concatenate(subsums, axis=0)
        carry = carry_ref[...]
        e_ref[...] = jnp.concatenate(
            [locs[j] + excl[j:j + 1, :] for j in range(nsub)],
            axis=0) + carry
        carry_ref[...] = carry + incl[nsub - 1:nsub, :]

    @pl.when(k == NB)
    def _():
        e_ref[...] = jnp.broadcast_to(carry_ref[...], (BS, D))
        b, g, inv = _boundary_logic(sim_scr[...], cu_ref)
        b_ref[...] = b
        g_ref[...] = g
        inv_ref[...] = inv


def _k12_call(flat, tri, cu_seqlens, interpret=False):
    return pl.pallas_call(
        _k12_body,
        grid=(NB + 1,),
        in_specs=[
            pl.BlockSpec((BS, D), lambda k: (jnp.minimum(k, NB - 1), 0)),
            pl.BlockSpec((SB, SB), lambda k: (0, 0)),
            pl.BlockSpec(memory_space=pltpu.SMEM),
        ],
        out_specs=[
            pl.BlockSpec((BS, D), lambda k: (k, 0)),
            pl.BlockSpec((128, 128), lambda k: (0, 0)),
            pl.BlockSpec((128, 128), lambda k: (0, 0)),
            pl.BlockSpec((128, 128), lambda k: (0, 0)),
        ],
        out_shape=[
            jax.ShapeDtypeStruct((N + BS, D), jnp.float32),
            jax.ShapeDtypeStruct((128, 128), jnp.int32),
            jax.ShapeDtypeStruct((128, 128), jnp.int32),
            jax.ShapeDtypeStruct((128, 128), jnp.float32),
        ],
        scratch_shapes=[
            pltpu.VMEM((1, D), jnp.float32),
            pltpu.VMEM((1, D), jnp.float32),
            pltpu.VMEM((128, 128), jnp.float32),
        ],
        interpret=interpret,
    )(flat, tri, cu_seqlens)


def _k3_body(e_hbm, b_hbm, g_hbm, inv_hbm, out_hbm,
             bi_v, gi_v, inv_v, rb_v, rg_v, sem_g, sem_s):
    wid = lax.axis_index("s") * SC_CORES + lax.axis_index("c")
    base = wid * ROWS_PER_WORKER

    def idx_copy(ci, p):
        off = base + ci * CH
        pltpu.sync_copy(b_hbm.at[pl.ds(off, CH)], bi_v[p])
        pltpu.sync_copy(g_hbm.at[pl.ds(off, CH)], gi_v[p])
        pltpu.sync_copy(inv_hbm.at[pl.ds(off, CH)], inv_v[p])

    def gather_start(ci, p):
        cb = pltpu.async_copy(e_hbm.at[bi_v[p]], rb_v[p], sem_g[p])
        cg = pltpu.async_copy(e_hbm.at[gi_v[p]], rg_v[p], sem_g[p])
        return cb, cg

    def combine(p):
        @plsc.parallel_loop(0, CH, unroll=2)
        def one_row(r):
            ivec = inv_v[p][r, :]
            for l in range(D // 16):
                sl = pl.ds(l * 16, 16)
                rg_v[p][r, sl] = (rg_v[p][r, sl] - rb_v[p][r, sl]) * ivec

    def store_start(ci, p):
        off = base + ci * CH
        return pltpu.async_copy(rg_v[p], out_hbm.at[pl.ds(off, CH)], sem_s[p])

    idx_copy(0, 0)
    gathers = {0: gather_start(0, 0)}
    stores = {}
    for ci in range(NCH):
        p = ci % 2
        if ci + 1 < NCH:
            idx_copy(ci + 1, 1 - p)
        for c in gathers.pop(ci):
            c.wait()
        if ci + 1 < NCH:
            if ci - 1 >= 0:
                stores.pop(ci - 1).wait()
            gathers[ci + 1] = gather_start(ci + 1, 1 - p)
        combine(p)
        stores[ci] = store_start(ci, p)
    for c in stores.values():
        c.wait()


def _k3_call(e, b, g, inv_rep):
    mesh = plsc.VectorSubcoreMesh(
        core_axis_name="c", subcore_axis_name="s",
        num_cores=SC_CORES, num_subcores=SC_SUBCORES)
    f = pl.kernel(
        _k3_body,
        out_type=jax.ShapeDtypeStruct((N, D), jnp.float32),
        mesh=mesh,
        scratch_types=[
            [pltpu.VMEM((CH,), jnp.int32) for _ in range(2)],
            [pltpu.VMEM((CH,), jnp.int32) for _ in range(2)],
            [pltpu.VMEM((CH, 16), jnp.float32) for _ in range(2)],
            [pltpu.VMEM((CH, D), jnp.float32) for _ in range(2)],
            [pltpu.VMEM((CH, D), jnp.float32) for _ in range(2)],
            [pltpu.SemaphoreType.DMA for _ in range(2)],
            [pltpu.SemaphoreType.DMA for _ in range(2)],
        ],
    )
    return f(e, b, g, inv_rep)


def kernel(flat, cu_seqlens):
    tri = jnp.tril(jnp.ones((SB, SB), jnp.float32), -1)
    e, b2, g2, inv2 = _k12_call(flat, tri, cu_seqlens)
    b = b2.reshape(N)
    g = g2.reshape(N)
    inv_rep = jnp.broadcast_to(inv2.reshape(N, 1), (N, 16))
    return _k3_call(e, b, g, inv_rep)


# speedup vs baseline: 1.0088x; 1.0088x over previous
import jax
import jax.numpy as jnp
import numpy as np
from jax import lax
from jax.experimental import pallas as pl
from jax.experimental.pallas import tpu as pltpu
from jax.experimental.pallas import tpu_sc as plsc

N = 16384
D = 256
BS = 2048
SB = 64
NB = N // BS
B_WIN = 20
L_SCALE = np.float32(0.5 * (0.95 + 1.05))


def _batcher_pairs(n):
    m = 1
    while m < n:
        m *= 2
    pairs = []

    def oddeven_merge(lo, hi, r):
        step = r * 2
        if step < hi - lo:
            oddeven_merge(lo, hi, step)
            oddeven_merge(lo + r, hi, step)
            for i in range(lo + r, hi - r, step):
                pairs.append((i, i + r))
        else:
            pairs.append((lo, lo + r))

    def oddeven_sort(lo, hi):
        if hi - lo >= 1:
            mid = lo + (hi - lo) // 2
            oddeven_sort(lo, mid)
            oddeven_sort(mid + 1, hi)
            oddeven_merge(lo, hi, 1)

    oddeven_sort(0, m - 1)
    return [(i, j) for (i, j) in pairs if j < n]


def _median_network(n):
    pairs = _batcher_pairs(n)
    needed = {n // 2 - 1, n // 2}
    keep = []
    for (i, j) in reversed(pairs):
        if i in needed or j in needed:
            keep.append((i, j))
            needed.add(i)
            needed.add(j)
    keep.reverse()
    return keep


_MED_NET = _median_network(2 * B_WIN)

SC_CORES = 2
SC_SUBCORES = 16
SC_WORKERS = SC_CORES * SC_SUBCORES
ROWS_PER_WORKER = N // SC_WORKERS
CH = 64
NCH = ROWS_PER_WORKER // CH


def _rows_down(z, q, fill):
    if q == 0:
        return z
    return jnp.concatenate(
        [jnp.full((q, 128), fill, z.dtype), z[:128 - q, :]], axis=0)


def _rows_up(z, q, fill):
    if q == 0:
        return z
    return jnp.concatenate(
        [z[q:, :], jnp.full((q, 128), fill, z.dtype)], axis=0)


def _shift_down(x, s, fill):
    q, b = s // 128, s % 128
    if b == 0:
        return _rows_down(x, q, fill)
    xq = _rows_down(x, q, fill)
    xq1 = _rows_down(x, q + 1, fill)
    lane = lax.broadcasted_iota(jnp.int32, (128, 128), 1)
    return jnp.where(lane >= b,
                     jnp.roll(xq, b, axis=1),
                     jnp.roll(xq1, b, axis=1))


def _shift_up(x, s, fill):
    q, b = s // 128, s % 128
    if b == 0:
        return _rows_up(x, q, fill)
    xq = _rows_up(x, q, fill)
    xq1 = _rows_up(x, q + 1, fill)
    lane = lax.broadcasted_iota(jnp.int32, (128, 128), 1)
    return jnp.where(lane < 128 - b,
                     jnp.roll(xq, -b, axis=1),
                     jnp.roll(xq1, -b, axis=1))


def _boundary_logic(sim_raw, cu_ref):
    row = lax.broadcasted_iota(jnp.int32, (128, 128), 0)
    lane = lax.broadcasted_iota(jnp.int32, (128, 128), 1)
    pos = row * 128 + lane

    cnt = jnp.zeros((128, 128), jnp.int32)
    for s in range(9):
        cnt = cnt + (cu_ref[s] <= pos).astype(jnp.int32)
    seq = jnp.clip(cnt - 1, 0, 7)
    starts = jnp.zeros((128, 128), jnp.int32)
    ends = jnp.zeros((128, 128), jnp.int32)
    for s in range(8):
        m = seq == s
        starts = jnp.where(m, cu_ref[s], starts)
        ends = jnp.where(m, cu_ref[s + 1] - 1, ends)
    ends = jnp.maximum(ends, starts)
    is_start = pos == starts
    sim = jnp.where(is_start, jnp.float32(1.0), sim_raw)

    sim_end = jnp.zeros((128, 128), jnp.float32)
    for s in range(8):
        e_s = cu_ref[s + 1] - 1
        val = jnp.sum(jnp.where(pos == e_s, sim, 0.0))
        sim_end = jnp.where(seq == s, val, sim_end)

    wins = []
    for off in range(-B_WIN, B_WIN):
        if off == 0:
            sh = sim
        elif off < 0:
            sh = _shift_down(sim, -off, 0.0)
        else:
            sh = _shift_up(sim, off, 0.0)
        ip = pos + off
        w = jnp.where(ip < starts, jnp.float32(1.0),
                      jnp.where(ip > ends, sim_end, sh))
        wins.append(w)

    W = 2 * B_WIN
    arr = list(wins)
    for (i, j) in _MED_NET:
        lo = jnp.minimum(arr[i], arr[j])
        hi = jnp.maximum(arr[i], arr[j])
        arr[i] = lo
        arr[j] = hi
    med = 0.5 * (arr[W // 2 - 1] + arr[W // 2])

    new_start = is_start | (sim < L_SCALE * med)

    x = jnp.where(new_start, pos, -1)
    s = 1
    while s < N:
        x = jnp.maximum(x, _shift_down(x, s, -1))
        s *= 2
    b = x
    y = jnp.where(new_start, pos, N)
    s = 1
    while s < N:
        y = jnp.minimum(y, _shift_up(y, s, N))
        s *= 2
    g = _shift_up(y, 1, N)
    inv = 1.0 / (g - b).astype(jnp.float32)
    return b, g, inv


def _k12_body(flat_ref, tri_ref, cu_ref, e_ref, b_ref, g_ref, inv_ref,
              carry_ref, prev_ref, sim_scr):
    k = pl.program_id(0)

    @pl.when(k == 0)
    def _():
        carry_ref[...] = jnp.zeros_like(carry_ref)

    @pl.when(k < NB)
    def _():
        blk = flat_ref[...]
        n = jnp.sqrt(jnp.sum(blk * blk, axis=1, keepdims=True))
        normed = blk / (n + 1e-8)
        prev = jnp.concatenate([prev_ref[...], normed[:-1, :]], axis=0)
        s = jnp.sum(normed * prev, axis=1)
        sim_scr[pl.ds(k * (BS // 128), BS // 128), :] = s.reshape(BS // 128, 128)
        prev_ref[...] = normed[BS - 1:, :]
        tri = tri_ref[...]
        locs = []
        subsums = []
        for j in range(BS // SB):
            sub = blk[j * SB:(j + 1) * SB, :]
            loc = lax.dot_general(
                tri, sub, (((1,), (0,)), ((), ())),
                preferred_element_type=jnp.float32,
                precision=lax.Precision.HIGHEST)
            locs.append(loc)
            subsums.append(loc[SB - 1:SB, :] + sub[SB - 1:SB, :])
        nsub = BS // SB
        incl = jnp.concatenate(subsums, axis=0)
        s2 = 1
        while s2 < nsub:
            incl = incl + jnp.concatenate(
                [jnp.zeros((s2, D), jnp.float32), incl[:nsub - s2, :]],
                axis=0)
            s2 *= 2
        excl = incl - jnp.concatenate(subsums, axis=0)
        carry = carry_ref[...]
        e_ref[...] = jnp.concatenate(
            [locs[j] + excl[j:j + 1, :] for j in range(nsub)],
            axis=0) + carry
        carry_ref[...] = carry + incl[nsub - 1:nsub, :]

    @pl.when(k == NB)
    def _():
        e_ref[...] = jnp.broadcast_to(carry_ref[...], (BS, D))
        b, g, inv = _boundary_logic(sim_scr[...], cu_ref)
        b_ref[...] = b
        g_ref[...] = g
        inv_ref[...] = inv


def _k12_call(flat, tri, cu_seqlens, interpret=False):
    return pl.pallas_call(
        _k12_body,
        grid=(NB + 1,),
        in_specs=[
            pl.BlockSpec((BS, D), lambda k: (jnp.minimum(k, NB - 1), 0)),
            pl.BlockSpec((SB, SB), lambda k: (0, 0)),
            pl.BlockSpec(memory_space=pltpu.SMEM),
        ],
        out_specs=[
            pl.BlockSpec((BS, D), lambda k: (k, 0)),
            pl.BlockSpec((128, 128), lambda k: (0, 0)),
            pl.BlockSpec((128, 128), lambda k: (0, 0)),
            pl.BlockSpec((128, 128), lambda k: (0, 0)),
        ],
        out_shape=[
            jax.ShapeDtypeStruct((N + BS, D), jnp.float32),
            jax.ShapeDtypeStruct((128, 128), jnp.int32),
            jax.ShapeDtypeStruct((128, 128), jnp.int32),
            jax.ShapeDtypeStruct((128, 128), jnp.float32),
        ],
        scratch_shapes=[
            pltpu.VMEM((1, D), jnp.float32),
            pltpu.VMEM((1, D), jnp.float32),
            pltpu.VMEM((128, 128), jnp.float32),
        ],
        interpret=interpret,
    )(flat, tri, cu_seqlens)


def _k3_body(e_hbm, b_hbm, g_hbm, inv_hbm, out_hbm,
             bi_v, gi_v, inv_v, rb_v, rg_v, sem_g, sem_s):
    wid = lax.axis_index("s") * SC_CORES + lax.axis_index("c")
    base = wid * ROWS_PER_WORKER

    def idx_copy(ci, p):
        off = base + ci * CH
        pltpu.sync_copy(b_hbm.at[pl.ds(off, CH)], bi_v[p])
        pltpu.sync_copy(g_hbm.at[pl.ds(off, CH)], gi_v[p])
        pltpu.sync_copy(inv_hbm.at[pl.ds(off, CH)], inv_v[p])

    def gather_start(ci, p):
        cb = pltpu.async_copy(e_hbm.at[bi_v[p]], rb_v[p], sem_g[p])
        cg = pltpu.async_copy(e_hbm.at[gi_v[p]], rg_v[p], sem_g[p])
        return cb, cg

    def combine(p):
        @plsc.parallel_loop(0, CH, unroll=2)
        def one_row(r):
            ivec = inv_v[p][r, :]
            for l in range(D // 16):
                sl = pl.ds(l * 16, 16)
                rg_v[p][r, sl] = (rg_v[p][r, sl] - rb_v[p][r, sl]) * ivec

    def store_start(ci, p):
        off = base + ci * CH
        return pltpu.async_copy(rg_v[p], out_hbm.at[pl.ds(off, CH)], sem_s[p])

    idx_copy(0, 0)
    gathers = {0: gather_start(0, 0)}
    stores = {}
    for ci in range(NCH):
        p = ci % 2
        if ci + 1 < NCH:
            idx_copy(ci + 1, 1 - p)
        for c in gathers.pop(ci):
            c.wait()
        if ci + 1 < NCH:
            if ci - 1 >= 0:
                stores.pop(ci - 1).wait()
            gathers[ci + 1] = gather_start(ci + 1, 1 - p)
        combine(p)
        stores[ci] = store_start(ci, p)
    for c in stores.values():
        c.wait()


def _k3_call(e, b, g, inv_rep):
    mesh = plsc.VectorSubcoreMesh(
        core_axis_name="c", subcore_axis_name="s",
        num_cores=SC_CORES, num_subcores=SC_SUBCORES)
    f = pl.kernel(
        _k3_body,
        out_type=jax.ShapeDtypeStruct((N, D), jnp.float32),
        mesh=mesh,
        scratch_types=[
            [pltpu.VMEM((CH,), jnp.int32) for _ in range(2)],
            [pltpu.VMEM((CH,), jnp.int32) for _ in range(2)],
            [pltpu.VMEM((CH, 16), jnp.float32) for _ in range(2)],
            [pltpu.VMEM((CH, D), jnp.float32) for _ in range(2)],
            [pltpu.VMEM((CH, D), jnp.float32) for _ in range(2)],
            [pltpu.SemaphoreType.DMA for _ in range(2)],
            [pltpu.SemaphoreType.DMA for _ in range(2)],
        ],
    )
    return f(e, b, g, inv_rep)


def kernel(flat, cu_seqlens):
    tri = jnp.tril(jnp.ones((SB, SB), jnp.float32), -1)
    e, b2, g2, inv2 = _k12_call(flat, tri, cu_seqlens)
    b = b2.reshape(N)
    g = g2.reshape(N)
    inv_rep = jnp.broadcast_to(inv2.reshape(N, 1), (N, 16))
    return _k3_call(e, b, g, inv_rep)
